# baseline (speedup 1.0000x reference)
import functools

import jax
import jax.numpy as jnp
from jax import lax
from jax.experimental import pallas as pl
from jax.experimental.pallas import tpu as pltpu
from jax.experimental.pallas import tpu_sc as plsc

N, C, W, H = 8, 96, 384, 384
RED = 96
OUT_W = 192

CB = 16

NC_SC = 2
NS_SC = 16
NWORK = NC_SC * NS_SC
PAIRS = N * C
PAIRS_PER_W = PAIRS // NWORK
G_CHUNK = 96
N_CHUNKS = OUT_W // G_CHUNK


def _maxpool_body(x_ref, o_ref):
    c = pl.program_id(1)
    m = jnp.max(x_ref[0], axis=(0, 2))[None, None, :]

    @pl.when(c == 0)
    def _():
        o_ref[...] = m

    @pl.when(c > 0)
    def _():
        o_ref[...] = jnp.maximum(o_ref[...], m)


def _max_pool(x):
    return pl.pallas_call(
        _maxpool_body,
        grid=(N, C // CB),
        in_specs=[pl.BlockSpec((1, CB, W, H), lambda n, c: (n, c, 0, 0))],
        out_specs=pl.BlockSpec((1, 1, W), lambda n, c: (n, 0, 0)),
        out_shape=jax.ShapeDtypeStruct((N, 1, W), jnp.float32),
    )(x).reshape(N, W)


def _mlp_topk_body(v_ref, w1_ref, b1_ref, g_ref, be_ref, w2_ref, b2_ref,
                   idx_ref):
    hi = lax.Precision.HIGHEST
    vv = v_ref[...]
    h = lax.dot_general(vv, w1_ref[...], (((1,), (1,)), ((), ())),
                        preferred_element_type=jnp.float32) + b1_ref[...]
    mean = jnp.mean(h, axis=0, keepdims=True)
    d = h - mean
    var = jnp.mean(d * d, axis=0, keepdims=True)
    hn = d / jnp.sqrt(var + 1e-5) * g_ref[...] + be_ref[...]
    hr = jnp.maximum(hn, 0.0)
    a = lax.dot_general(hr, w2_ref[...], (((1,), (1,)), ((), ())),
                        preferred_element_type=jnp.float32) + b2_ref[...]

    ii_l = lax.broadcasted_iota(jnp.int32, (W, W), 1)
    jj_s = lax.broadcasted_iota(jnp.int32, (W, W), 0)
    eye = (ii_l == jj_s).astype(jnp.float32)
    u_strict = (jj_s < ii_l).astype(jnp.float32)
    at = lax.dot_general(eye, a, (((1,), (1,)), ((), ())),
                         precision=hi, preferred_element_type=jnp.float32)
    irow = lax.broadcasted_iota(jnp.int32, (1, W), 1).astype(jnp.float32)
    kk_l = lax.broadcasted_iota(jnp.int32, (W, OUT_W), 1).astype(jnp.float32)

    for n in range(N):
        ai = a[n:n + 1, :]
        aj = at[:, n:n + 1]
        beats = (aj > ai) | ((aj == ai) & (jj_s < ii_l))
        rank = jnp.sum(beats.astype(jnp.float32), axis=0)[None, :]
        sel = (rank < float(OUT_W)).astype(jnp.float32)
        pos = lax.dot_general(sel, u_strict, (((1,), (0,)), ((), ())),
                              precision=hi,
                              preferred_element_type=jnp.float32)
        pc = lax.dot_general(eye, pos, (((1,), (1,)), ((), ())),
                             precision=hi,
                             preferred_element_type=jnp.float32)
        sc = lax.dot_general(eye, sel, (((1,), (1,)), ((), ())),
                             precision=hi,
                             preferred_element_type=jnp.float32)
        onehot = jnp.where((pc == kk_l) & (sc > 0.5), 1.0, 0.0)
        idxrow = lax.dot_general(irow, onehot, (((1,), (0,)), ((), ())),
                                 precision=hi,
                                 preferred_element_type=jnp.float32)
        idx_ref[n:n + 1, :] = idxrow.astype(jnp.int32)


def _mlp_topk(v, W1, b1, gamma, beta, W2, b2):
    return pl.pallas_call(
        _mlp_topk_body,
        out_shape=jax.ShapeDtypeStruct((N, OUT_W), jnp.int32),
    )(v, W1, b1.reshape(1, RED), gamma.reshape(1, RED), beta.reshape(1, RED),
      W2, b2.reshape(1, W))


NBUF = 3
CHUNKS_PER_W = PAIRS_PER_W * N_CHUNKS
SUPERS = CHUNKS_PER_W // NBUF


def _sc_gather_body(x_hbm, idx_hbm, out_hbm, idxraw_v, idxc_v, rows_v,
                    gs0, gs1, gs2, ss0, ss1, ss2):
    cid = lax.axis_index("c")
    sid = lax.axis_index("s")
    wid = sid * NC_SC + cid
    gsems = [gs0, gs1, gs2]
    ssems = [ss0, ss1, ss2]

    n_w = wid // (C // PAIRS_PER_W)
    pltpu.sync_copy(idx_hbm.at[n_w], idxraw_v)

    def super_body(s, carry):
        gathers = []
        metas = []
        for bb in range(NBUF):
            g = s * NBUF + bb
            j = g // N_CHUNKS
            q = g % N_CHUNKS
            p = wid * PAIRS_PER_W + j
            base = p * W

            @pl.when(s > 0)
            def _():
                pltpu.make_async_copy(
                    rows_v.at[bb], out_hbm.at[pl.ds(0, G_CHUNK)],
                    ssems[bb]).wait()

            for t in range(G_CHUNK // 16):
                idxc_v[bb, pl.ds(t * 16, 16)] = (
                    idxraw_v[pl.ds(q * G_CHUNK + t * 16, 16)] + base)
            gathers.append(pltpu.async_copy(
                x_hbm.at[idxc_v.at[bb]], rows_v.at[bb], gsems[bb]))
            metas.append(p * OUT_W + q * G_CHUNK)
        for bb in range(NBUF):
            gathers[bb].wait()
            pltpu.async_copy(rows_v.at[bb],
                             out_hbm.at[pl.ds(metas[bb], G_CHUNK)],
                             ssems[bb])
        return carry

    lax.fori_loop(0, SUPERS, super_body, 0)
    for bb in range(NBUF):
        pltpu.make_async_copy(rows_v.at[bb], out_hbm.at[pl.ds(0, G_CHUNK)],
                              ssems[bb]).wait()


def _sc_gather(x2d, idx):
    mesh = plsc.VectorSubcoreMesh(core_axis_name="c", subcore_axis_name="s")
    run = functools.partial(
        pl.kernel,
        out_type=jax.ShapeDtypeStruct((PAIRS * OUT_W, H), jnp.float32),
        mesh=mesh,
        scratch_types=[
            pltpu.VMEM((OUT_W,), jnp.int32),
            pltpu.VMEM((NBUF, G_CHUNK), jnp.int32),
            pltpu.VMEM((NBUF, G_CHUNK, H), jnp.float32),
            pltpu.SemaphoreType.DMA,
            pltpu.SemaphoreType.DMA,
            pltpu.SemaphoreType.DMA,
            pltpu.SemaphoreType.DMA,
            pltpu.SemaphoreType.DMA,
            pltpu.SemaphoreType.DMA,
        ],
    )(_sc_gather_body)
    return run(x2d, idx)


def kernel(x, W1, b1, gamma, beta, W2, b2):
    v = _max_pool(x)
    idx = _mlp_topk(v, W1, b1, gamma, beta, W2, b2)
    pooled = _sc_gather(x.reshape(N * C * W, H), idx)
    return pooled.reshape(N, C, OUT_W, H)


# speedup vs baseline: 1.0127x; 1.0127x over previous
import functools

import jax
import jax.numpy as jnp
from jax import lax
from jax.experimental import pallas as pl
from jax.experimental.pallas import tpu as pltpu
from jax.experimental.pallas import tpu_sc as plsc

N, C, W, H = 8, 96, 384, 384
RED = 96
OUT_W = 192

CB = 16

NC_SC = 2
NS_SC = 16
NWORK = NC_SC * NS_SC
PAIRS = N * C
PAIRS_PER_W = PAIRS // NWORK
G_CHUNK = 96
N_CHUNKS = OUT_W // G_CHUNK


def _maxpool_body(x_ref, o_ref):
    c = pl.program_id(1)
    m = jnp.max(x_ref[0], axis=(0, 2))[None, None, :]

    @pl.when(c == 0)
    def _():
        o_ref[...] = m

    @pl.when(c > 0)
    def _():
        o_ref[...] = jnp.maximum(o_ref[...], m)


def _max_pool(x):
    return pl.pallas_call(
        _maxpool_body,
        grid=(N, C // CB),
        in_specs=[pl.BlockSpec((1, CB, W, H), lambda n, c: (n, c, 0, 0))],
        out_specs=pl.BlockSpec((1, 1, W), lambda n, c: (n, 0, 0)),
        out_shape=jax.ShapeDtypeStruct((N, 1, W), jnp.float32),
    )(x).reshape(N, W)


def _mlp_topk_body(v_ref, w1_ref, b1_ref, g_ref, be_ref, w2_ref, b2_ref,
                   idx_ref):
    hi = lax.Precision.HIGHEST
    vv = v_ref[...]
    h = lax.dot_general(vv, w1_ref[...], (((1,), (1,)), ((), ())),
                        preferred_element_type=jnp.float32) + b1_ref[...]
    mean = jnp.mean(h, axis=0, keepdims=True)
    d = h - mean
    var = jnp.mean(d * d, axis=0, keepdims=True)
    hn = d / jnp.sqrt(var + 1e-5) * g_ref[...] + be_ref[...]
    hr = jnp.maximum(hn, 0.0)
    a = lax.dot_general(hr, w2_ref[...], (((1,), (1,)), ((), ())),
                        preferred_element_type=jnp.float32) + b2_ref[...]

    ii_l = lax.broadcasted_iota(jnp.int32, (W, W), 1)
    jj_s = lax.broadcasted_iota(jnp.int32, (W, W), 0)
    eye = (ii_l == jj_s).astype(jnp.float32)
    u_strict = (jj_s < ii_l).astype(jnp.float32)
    at = lax.dot_general(eye, a, (((1,), (1,)), ((), ())),
                         precision=hi, preferred_element_type=jnp.float32)
    irow = lax.broadcasted_iota(jnp.int32, (1, W), 1).astype(jnp.float32)
    kk_l = lax.broadcasted_iota(jnp.int32, (W, OUT_W), 1).astype(jnp.float32)

    for n in range(N):
        ai = a[n:n + 1, :]
        aj = at[:, n:n + 1]
        beats = (aj > ai) | ((aj == ai) & (jj_s < ii_l))
        rank = jnp.sum(beats.astype(jnp.float32), axis=0)[None, :]
        sel = (rank < float(OUT_W)).astype(jnp.float32)
        pos = lax.dot_general(sel, u_strict, (((1,), (0,)), ((), ())),
                              precision=hi,
                              preferred_element_type=jnp.float32)
        pc = lax.dot_general(eye, pos, (((1,), (1,)), ((), ())),
                             precision=hi,
                             preferred_element_type=jnp.float32)
        sc = lax.dot_general(eye, sel, (((1,), (1,)), ((), ())),
                             precision=hi,
                             preferred_element_type=jnp.float32)
        onehot = jnp.where((pc == kk_l) & (sc > 0.5), 1.0, 0.0)
        idxrow = lax.dot_general(irow, onehot, (((1,), (0,)), ((), ())),
                                 precision=hi,
                                 preferred_element_type=jnp.float32)
        idx_ref[n:n + 1, :] = idxrow.astype(jnp.int32)


def _mlp_topk(v, W1, b1, gamma, beta, W2, b2):
    return pl.pallas_call(
        _mlp_topk_body,
        out_shape=jax.ShapeDtypeStruct((N, OUT_W), jnp.int32),
    )(v, W1, b1.reshape(1, RED), gamma.reshape(1, RED), beta.reshape(1, RED),
      W2, b2.reshape(1, W))


NBUF = 3
CHUNKS_PER_W = PAIRS_PER_W * N_CHUNKS
SUPERS = CHUNKS_PER_W // NBUF


def _sc_gather_body(x_hbm, idx_hbm, out_hbm, idxraw_v, idxc_v, rows_v,
                    gs0, gs1, gs2, ss0, ss1, ss2):
    cid = lax.axis_index("c")
    sid = lax.axis_index("s")
    wid = sid * NC_SC + cid
    gsems = [gs0, gs1, gs2]
    ssems = [ss0, ss1, ss2]

    n_w = wid // (C // PAIRS_PER_W)
    pltpu.sync_copy(idx_hbm.at[n_w], idxraw_v)

    def out_off(g):
        j = g // N_CHUNKS
        q = g % N_CHUNKS
        return (wid * PAIRS_PER_W + j) * OUT_W + q * G_CHUNK

    def fire_gather(g, bb):
        j = g // N_CHUNKS
        q = g % N_CHUNKS
        base = (wid * PAIRS_PER_W + j) * W
        for t in range(G_CHUNK // 16):
            idxc_v[bb, pl.ds(t * 16, 16)] = (
                idxraw_v[pl.ds(q * G_CHUNK + t * 16, 16)] + base)
        pltpu.async_copy(x_hbm.at[idxc_v.at[bb]], rows_v.at[bb], gsems[bb])

    def drain_gather(bb):
        pltpu.make_async_copy(x_hbm.at[idxc_v.at[bb]], rows_v.at[bb],
                              gsems[bb]).wait()

    def drain_store(bb):
        pltpu.make_async_copy(rows_v.at[bb], out_hbm.at[pl.ds(0, G_CHUNK)],
                              ssems[bb]).wait()

    def fire_store(g, bb):
        pltpu.async_copy(rows_v.at[bb], out_hbm.at[pl.ds(out_off(g), G_CHUNK)],
                         ssems[bb])

    def super_body(s, carry):
        for bb in range(NBUF):
            g = s * NBUF + bb

            @pl.when(s > 0)
            def _():
                drain_store(bb)

            fire_gather(g, bb)
            pb = (bb + NBUF - 1) % NBUF
            if bb == 0:
                @pl.when(s > 0)
                def _():
                    drain_gather(pb)
                    fire_store(g - 1, pb)
            else:
                drain_gather(pb)
                fire_store(g - 1, pb)
        return carry

    lax.fori_loop(0, SUPERS, super_body, 0)
    last = CHUNKS_PER_W - 1
    lb = last % NBUF
    drain_gather(lb)
    fire_store(last, lb)
    for bb in range(NBUF):
        drain_store(bb)


def _sc_gather(x2d, idx):
    mesh = plsc.VectorSubcoreMesh(core_axis_name="c", subcore_axis_name="s")
    run = functools.partial(
        pl.kernel,
        out_type=jax.ShapeDtypeStruct((PAIRS * OUT_W, H), jnp.float32),
        mesh=mesh,
        scratch_types=[
            pltpu.VMEM((OUT_W,), jnp.int32),
            pltpu.VMEM((NBUF, G_CHUNK), jnp.int32),
            pltpu.VMEM((NBUF, G_CHUNK, H), jnp.float32),
            pltpu.SemaphoreType.DMA,
            pltpu.SemaphoreType.DMA,
            pltpu.SemaphoreType.DMA,
            pltpu.SemaphoreType.DMA,
            pltpu.SemaphoreType.DMA,
            pltpu.SemaphoreType.DMA,
        ],
    )(_sc_gather_body)
    return run(x2d, idx)


def kernel(x, W1, b1, gamma, beta, W2, b2):
    v = _max_pool(x)
    idx = _mlp_topk(v, W1, b1, gamma, beta, W2, b2)
    pooled = _sc_gather(x.reshape(N * C * W, H), idx)
    return pooled.reshape(N, C, OUT_W, H)
